# baseline (speedup 1.0000x reference)
import functools

import jax
import jax.numpy as jnp
from jax import lax
from jax.experimental import pallas as pl
from jax.experimental.pallas import tpu as pltpu
from jax.experimental.pallas import tpu_sc as plsc

N_NODES = 10000
N_PAD = 10240
D = 128
NC = 2
NS = 16
NT = NC * NS
LANES = 16
CH = 128
SUP = 8

ROW_STEP = 624
ROW_BUF = 640

_MESH = plsc.VectorSubcoreMesh(core_axis_name="c", subcore_axis_name="s")


def _sc_degree(dst2, dst_tail):
    n_ch = dst2.shape[0] - (dst2.shape[0] % SUP)
    n_sup = n_ch // SUP
    base_sup, extra_sup = divmod(n_sup, NT)
    tail_ch = dst_tail.shape[0]
    assert tail_ch <= NT

    @functools.partial(
        pl.kernel,
        out_type=jax.ShapeDtypeStruct((NT, N_PAD // 128, 128), jnp.float32),
        mesh=_MESH,
        scratch_types=[
            pltpu.VMEM((2, SUP, CH), jnp.int32),
            pltpu.VMEM((tail_ch, CH), jnp.int32),
            pltpu.VMEM((N_PAD // 128, 128), jnp.float32),
            pltpu.SemaphoreType.DMA,
        ],
        compiler_params=pltpu.CompilerParams(needs_layout_passes=False),
    )
    def k(dst_hbm, dtail_hbm, out_hbm, didx, dtail, hist, ksem):
        c = lax.axis_index("c")
        s = lax.axis_index("s")
        t = c * NS + s
        n_mine = base_sup + jnp.where(t < extra_sup, 1, 0)

        zv = jnp.zeros((LANES,), jnp.float32)

        def zfill(r, _):
            for j in range(128 // LANES):
                hist[r, pl.ds(j * LANES, LANES)] = zv
            return 0

        lax.fori_loop(0, N_PAD // 128, zfill, 0)

        ov = jnp.ones((LANES,), jnp.float32)

        def scatter_rows(idx_ref2d, j):
            for v in range(CH // LANES):
                idx = idx_ref2d[j, pl.ds(v * LANES, LANES)]
                ihi = lax.shift_right_logical(idx, 7)
                ilo = lax.bitwise_and(idx, 127)
                plsc.addupdate_scatter(hist, [ihi, ilo], ov)

        @pl.when(n_mine > 0)
        def _():
            pltpu.async_copy(dst_hbm.at[pl.ds(t * SUP, SUP)], didx.at[0], ksem)

        def sup_body(u, _):
            b = lax.rem(u, 2)
            pltpu.make_async_copy(dst_hbm.at[pl.ds(0, SUP)], didx.at[b],
                                  ksem).wait()

            @pl.when(u + 1 < n_mine)
            def _():
                row = (t + (u + 1) * NT) * SUP
                pltpu.async_copy(dst_hbm.at[pl.ds(row, SUP)], didx.at[1 - b],
                                 ksem)

            for j in range(SUP):
                scatter_rows(didx.at[b], j)
            return 0

        lax.fori_loop(0, n_mine, sup_body, 0)

        if tail_ch:
            @pl.when(t < tail_ch)
            def _():
                pltpu.sync_copy(dtail_hbm, dtail)
                scatter_rows(dtail, t)

        pltpu.sync_copy(hist, out_hbm.at[t])

    return k(dst2, dst_tail)


def _tc_matmul(x, w):
    n = x.shape[0]
    d_out = w.shape[1]

    def body(x_ref, w_ref, h_ref):
        h_ref[...] = jnp.dot(x_ref[...], w_ref[...],
                             preferred_element_type=jnp.float32)

    return pl.pallas_call(
        body,
        out_shape=jax.ShapeDtypeStruct((n, d_out), jnp.float32),
    )(x, w)


def _tc_scale(h, hist):
    n = h.shape[0]

    def body(h_ref, hist_ref, g_ref, recip_ref):
        deg = jnp.sum(hist_ref[...], axis=0).reshape(N_PAD) + 1.0
        dis_col = lax.rsqrt(deg)[:, None]
        recip_ref[...] = (1.0 / deg)[:, None]
        g_ref[...] = h_ref[...] * dis_col[0:n]

    return pl.pallas_call(
        body,
        out_shape=[
            jax.ShapeDtypeStruct(h.shape, jnp.float32),
            jax.ShapeDtypeStruct((N_PAD, 1), jnp.float32),
        ],
    )(h, hist)


def _sc_aggregate(g, src2, dst2, src_tail, dst_tail, zeros_rows):
    n_ch = src2.shape[0] - (src2.shape[0] % SUP)
    n_sup = n_ch // SUP
    base_sup, extra_sup = divmod(n_sup, NT)
    tail_ch = src_tail.shape[0]

    @functools.partial(
        pl.kernel,
        out_type=jax.ShapeDtypeStruct((NC, N_NODES, D), jnp.float32),
        mesh=_MESH,
        scratch_types=[
            pltpu.VMEM((2, SUP, CH), jnp.int32),
            pltpu.VMEM((2, SUP, CH), jnp.int32),
            pltpu.VMEM((2, CH, D), jnp.float32),
            pltpu.VMEM((tail_ch, CH), jnp.int32),
            pltpu.VMEM((tail_ch, CH), jnp.int32),
            pltpu.VMEM_SHARED((N_NODES, D), jnp.float32),
            pltpu.SemaphoreType.DMA,
            pltpu.SemaphoreType.DMA,
            pltpu.SemaphoreType.DMA,
            pltpu.SemaphoreType.DMA,
            pltpu.SemaphoreType.DMA,
        ],
    )
    def k(g_hbm, src_hbm, dst_hbm, stail_hbm, dtail_hbm, zer_hbm, out_hbm,
          sidx, didx, rows, stail, dtail, acc_sp, ksem,
          gsem0, gsem1, ssem0, ssem1):
        c = lax.axis_index("c")
        s = lax.axis_index("s")
        t = c * NS + s
        n_mine = base_sup + jnp.where(t < extra_sup, 1, 0)
        gsems = (gsem0, gsem1)
        ssems = (ssem0, ssem1)

        r0 = s * ROW_STEP

        @pl.when(c == 0)
        def _():
            pltpu.sync_copy(g_hbm.at[pl.ds(r0, ROW_BUF)],
                            acc_sp.at[pl.ds(r0, ROW_BUF)])

        @pl.when(c == 1)
        def _():
            pltpu.sync_copy(zer_hbm, acc_sp.at[pl.ds(r0, ROW_BUF)])

        plsc.subcore_barrier()

        @pl.when(n_mine > 0)
        def _():
            pltpu.async_copy(src_hbm.at[pl.ds(t * SUP, SUP)], sidx.at[0], ksem)
            pltpu.async_copy(dst_hbm.at[pl.ds(t * SUP, SUP)], didx.at[0], ksem)

        def sup_body(u, _):
            b = lax.rem(u, 2)
            pltpu.make_async_copy(src_hbm.at[pl.ds(0, SUP)], sidx.at[b],
                                  ksem).wait()
            pltpu.make_async_copy(dst_hbm.at[pl.ds(0, SUP)], didx.at[b],
                                  ksem).wait()

            @pl.when(u + 1 < n_mine)
            def _():
                row = (t + (u + 1) * NT) * SUP
                pltpu.async_copy(src_hbm.at[pl.ds(row, SUP)], sidx.at[1 - b],
                                 ksem)
                pltpu.async_copy(dst_hbm.at[pl.ds(row, SUP)], didx.at[1 - b],
                                 ksem)

            gdesc = [None, None]
            sdesc = [None, None]
            gdesc[0] = pltpu.async_copy(g_hbm.at[sidx.at[b, 0]], rows.at[0],
                                        gsems[0])
            for j in range(SUP):
                r2 = j & 1
                n2 = 1 - r2
                if j >= 1:
                    sdesc[n2].wait()
                if j + 1 < SUP:
                    gdesc[n2] = pltpu.async_copy(
                        g_hbm.at[sidx.at[b, j + 1]], rows.at[n2], gsems[n2])
                gdesc[r2].wait()
                sdesc[r2] = pltpu.async_copy(
                    rows.at[r2], acc_sp.at[didx.at[b, j]], ssems[r2],
                    add=True)
            sdesc[(SUP - 1) & 1].wait()
            return 0

        lax.fori_loop(0, n_mine, sup_body, 0)

        if tail_ch:
            @pl.when(t < tail_ch)
            def _():
                pltpu.sync_copy(stail_hbm, stail)
                pltpu.sync_copy(dtail_hbm, dtail)
                pltpu.async_copy(g_hbm.at[stail.at[t]], rows.at[0],
                                 gsem0).wait()
                pltpu.sync_copy(rows.at[0], acc_sp.at[dtail.at[t]], add=True)

        plsc.subcore_barrier()

        pltpu.sync_copy(acc_sp.at[pl.ds(r0, ROW_BUF)],
                        out_hbm.at[c].at[pl.ds(r0, ROW_BUF)])

    return k(g, src2, dst2, src_tail, dst_tail, zeros_rows)


def _tc_finalize(acc2, recip, b):
    _, n, d = acc2.shape

    def body(a_ref, r_ref, b_ref, o_ref):
        v = (a_ref[0] + a_ref[1]) * r_ref[0:n] + b_ref[...][None, :]
        o_ref[...] = v / (1.0 + jnp.exp(-v))

    return pl.pallas_call(
        body,
        out_shape=jax.ShapeDtypeStruct((n, d), jnp.float32),
    )(acc2, recip, b)


def kernel(x, edge_index, W, b):
    src2 = edge_index[0].astype(jnp.int32).reshape(-1, CH)
    dst2 = edge_index[1].astype(jnp.int32).reshape(-1, CH)
    n_full = (src2.shape[0] // SUP) * SUP
    src_tail = src2[n_full:]
    dst_tail = dst2[n_full:]
    hist = _sc_degree(dst2, dst_tail)
    h = _tc_matmul(x, W)
    g, recip = _tc_scale(h, hist)
    acc2 = _sc_aggregate(g, src2, dst2, src_tail, dst_tail,
                         jnp.zeros((ROW_BUF, D), jnp.float32))
    return _tc_finalize(acc2, recip, b)


# speedup vs baseline: 1.0078x; 1.0078x over previous
import functools

import jax
import jax.numpy as jnp
from jax import lax
from jax.experimental import pallas as pl
from jax.experimental.pallas import tpu as pltpu
from jax.experimental.pallas import tpu_sc as plsc

N_NODES = 10000
N_PAD = 10240
D = 128
NC = 2
NS = 16
NT = NC * NS
LANES = 16
CH = 128
SUP = 8

ROW_STEP = 624
ROW_BUF = 640

_MESH = plsc.VectorSubcoreMesh(core_axis_name="c", subcore_axis_name="s")


def _sc_degree(dst2, dst_tail):
    n_ch = dst2.shape[0] - (dst2.shape[0] % SUP)
    n_sup = n_ch // SUP
    base_sup, extra_sup = divmod(n_sup, NT)
    tail_ch = dst_tail.shape[0]
    assert tail_ch <= NT

    @functools.partial(
        pl.kernel,
        out_type=jax.ShapeDtypeStruct((NT, N_PAD // 128, 128), jnp.float32),
        mesh=_MESH,
        scratch_types=[
            pltpu.VMEM((2, SUP, CH), jnp.int32),
            pltpu.VMEM((tail_ch, CH), jnp.int32),
            pltpu.VMEM((N_PAD // 128, 128), jnp.float32),
            pltpu.SemaphoreType.DMA,
        ],
        compiler_params=pltpu.CompilerParams(needs_layout_passes=False),
    )
    def k(dst_hbm, dtail_hbm, out_hbm, didx, dtail, hist, ksem):
        c = lax.axis_index("c")
        s = lax.axis_index("s")
        t = c * NS + s
        n_mine = base_sup + jnp.where(t < extra_sup, 1, 0)

        zv = jnp.zeros((LANES,), jnp.float32)

        def zfill(r, _):
            for j in range(128 // LANES):
                hist[r, pl.ds(j * LANES, LANES)] = zv
            return 0

        lax.fori_loop(0, N_PAD // 128, zfill, 0)

        ov = jnp.ones((LANES,), jnp.float32)

        def scatter_rows(idx_ref2d, j):
            for v in range(CH // LANES):
                idx = idx_ref2d[j, pl.ds(v * LANES, LANES)]
                ihi = lax.shift_right_logical(idx, 7)
                ilo = lax.bitwise_and(idx, 127)
                plsc.addupdate_scatter(hist, [ihi, ilo], ov)

        @pl.when(n_mine > 0)
        def _():
            pltpu.async_copy(dst_hbm.at[pl.ds(t * SUP, SUP)], didx.at[0], ksem)

        def sup_body(u, _):
            b = lax.rem(u, 2)
            pltpu.make_async_copy(dst_hbm.at[pl.ds(0, SUP)], didx.at[b],
                                  ksem).wait()

            @pl.when(u + 1 < n_mine)
            def _():
                row = (t + (u + 1) * NT) * SUP
                pltpu.async_copy(dst_hbm.at[pl.ds(row, SUP)], didx.at[1 - b],
                                 ksem)

            for j in range(SUP):
                scatter_rows(didx.at[b], j)
            return 0

        lax.fori_loop(0, n_mine, sup_body, 0)

        if tail_ch:
            @pl.when(t < tail_ch)
            def _():
                pltpu.sync_copy(dtail_hbm, dtail)
                scatter_rows(dtail, t)

        pltpu.sync_copy(hist, out_hbm.at[t])

    return k(dst2, dst_tail)


def _tc_dense(x, w, hist):
    n = x.shape[0]
    d_out = w.shape[1]

    def body(x_ref, w_ref, hist_ref, g_ref, recip_ref):
        deg = jnp.sum(hist_ref[...], axis=0).reshape(N_PAD) + 1.0
        dis_col = lax.rsqrt(deg)[:, None]
        recip_ref[...] = (1.0 / deg)[:, None]
        h = jnp.dot(x_ref[...], w_ref[...], preferred_element_type=jnp.float32)
        g_ref[...] = h * dis_col[0:n]

    return pl.pallas_call(
        body,
        out_shape=[
            jax.ShapeDtypeStruct((n, d_out), jnp.float32),
            jax.ShapeDtypeStruct((N_PAD, 1), jnp.float32),
        ],
    )(x, w, hist)


def _sc_aggregate(g, src2, dst2, src_tail, dst_tail, zeros_rows):
    n_ch = src2.shape[0] - (src2.shape[0] % SUP)
    n_sup = n_ch // SUP
    base_sup, extra_sup = divmod(n_sup, NT)
    tail_ch = src_tail.shape[0]

    @functools.partial(
        pl.kernel,
        out_type=jax.ShapeDtypeStruct((NC, N_NODES, D), jnp.float32),
        mesh=_MESH,
        scratch_types=[
            pltpu.VMEM((2, SUP, CH), jnp.int32),
            pltpu.VMEM((2, SUP, CH), jnp.int32),
            pltpu.VMEM((2, CH, D), jnp.float32),
            pltpu.VMEM((tail_ch, CH), jnp.int32),
            pltpu.VMEM((tail_ch, CH), jnp.int32),
            pltpu.VMEM_SHARED((N_NODES, D), jnp.float32),
            pltpu.SemaphoreType.DMA,
            pltpu.SemaphoreType.DMA,
            pltpu.SemaphoreType.DMA,
            pltpu.SemaphoreType.DMA,
            pltpu.SemaphoreType.DMA,
        ],
    )
    def k(g_hbm, src_hbm, dst_hbm, stail_hbm, dtail_hbm, zer_hbm, out_hbm,
          sidx, didx, rows, stail, dtail, acc_sp, ksem,
          gsem0, gsem1, ssem0, ssem1):
        c = lax.axis_index("c")
        s = lax.axis_index("s")
        t = c * NS + s
        n_mine = base_sup + jnp.where(t < extra_sup, 1, 0)
        gsems = (gsem0, gsem1)
        ssems = (ssem0, ssem1)

        r0 = s * ROW_STEP

        @pl.when(c == 0)
        def _():
            pltpu.sync_copy(g_hbm.at[pl.ds(r0, ROW_BUF)],
                            acc_sp.at[pl.ds(r0, ROW_BUF)])

        @pl.when(c == 1)
        def _():
            pltpu.sync_copy(zer_hbm, acc_sp.at[pl.ds(r0, ROW_BUF)])

        plsc.subcore_barrier()

        @pl.when(n_mine > 0)
        def _():
            pltpu.async_copy(src_hbm.at[pl.ds(t * SUP, SUP)], sidx.at[0], ksem)
            pltpu.async_copy(dst_hbm.at[pl.ds(t * SUP, SUP)], didx.at[0], ksem)

        def sup_body(u, _):
            b = lax.rem(u, 2)
            pltpu.make_async_copy(src_hbm.at[pl.ds(0, SUP)], sidx.at[b],
                                  ksem).wait()
            pltpu.make_async_copy(dst_hbm.at[pl.ds(0, SUP)], didx.at[b],
                                  ksem).wait()

            @pl.when(u + 1 < n_mine)
            def _():
                row = (t + (u + 1) * NT) * SUP
                pltpu.async_copy(src_hbm.at[pl.ds(row, SUP)], sidx.at[1 - b],
                                 ksem)
                pltpu.async_copy(dst_hbm.at[pl.ds(row, SUP)], didx.at[1 - b],
                                 ksem)

            gdesc = [None, None]
            sdesc = [None, None]
            gdesc[0] = pltpu.async_copy(g_hbm.at[sidx.at[b, 0]], rows.at[0],
                                        gsems[0])
            for j in range(SUP):
                r2 = j & 1
                n2 = 1 - r2
                if j >= 1:
                    sdesc[n2].wait()
                if j + 1 < SUP:
                    gdesc[n2] = pltpu.async_copy(
                        g_hbm.at[sidx.at[b, j + 1]], rows.at[n2], gsems[n2])
                gdesc[r2].wait()
                sdesc[r2] = pltpu.async_copy(
                    rows.at[r2], acc_sp.at[didx.at[b, j]], ssems[r2],
                    add=True)
            sdesc[(SUP - 1) & 1].wait()
            return 0

        lax.fori_loop(0, n_mine, sup_body, 0)

        if tail_ch:
            @pl.when(t < tail_ch)
            def _():
                pltpu.sync_copy(stail_hbm, stail)
                pltpu.sync_copy(dtail_hbm, dtail)
                pltpu.async_copy(g_hbm.at[stail.at[t]], rows.at[0],
                                 gsem0).wait()
                pltpu.sync_copy(rows.at[0], acc_sp.at[dtail.at[t]], add=True)

        plsc.subcore_barrier()

        pltpu.sync_copy(acc_sp.at[pl.ds(r0, ROW_BUF)],
                        out_hbm.at[c].at[pl.ds(r0, ROW_BUF)])

    return k(g, src2, dst2, src_tail, dst_tail, zeros_rows)


def _tc_finalize(acc2, recip, b):
    _, n, d = acc2.shape

    def body(a_ref, r_ref, b_ref, o_ref):
        v = (a_ref[0] + a_ref[1]) * r_ref[0:n] + b_ref[...][None, :]
        o_ref[...] = v / (1.0 + jnp.exp(-v))

    return pl.pallas_call(
        body,
        out_shape=jax.ShapeDtypeStruct((n, d), jnp.float32),
    )(acc2, recip, b)


def kernel(x, edge_index, W, b):
    src2 = edge_index[0].astype(jnp.int32).reshape(-1, CH)
    dst2 = edge_index[1].astype(jnp.int32).reshape(-1, CH)
    n_full = (src2.shape[0] // SUP) * SUP
    src_tail = src2[n_full:]
    dst_tail = dst2[n_full:]
    hist = _sc_degree(dst2, dst_tail)
    g, recip = _tc_dense(x, W, hist)
    acc2 = _sc_aggregate(g, src2, dst2, src_tail, dst_tail,
                         jnp.zeros((ROW_BUF, D), jnp.float32))
    return _tc_finalize(acc2, recip, b)


# speedup vs baseline: 1.0564x; 1.0483x over previous
import functools

import jax
import jax.numpy as jnp
from jax import lax
from jax.experimental import pallas as pl
from jax.experimental.pallas import tpu as pltpu
from jax.experimental.pallas import tpu_sc as plsc

N_NODES = 10000
N_PAD = 10240
D = 128
NC = 2
NS = 16
NT = NC * NS
LANES = 16
CH = 128
SUP = 8

ROW_STEP = 624
ROW_BUF = 640

_MESH = plsc.VectorSubcoreMesh(core_axis_name="c", subcore_axis_name="s")


def _sc_degree(dst2, dst_tail):
    n_ch = dst2.shape[0] - (dst2.shape[0] % SUP)
    n_sup = n_ch // SUP
    base_sup, extra_sup = divmod(n_sup, NT)
    tail_ch = dst_tail.shape[0]
    assert tail_ch <= NT

    @functools.partial(
        pl.kernel,
        out_type=jax.ShapeDtypeStruct((NT, N_PAD // 128, 128), jnp.float32),
        mesh=_MESH,
        scratch_types=[
            pltpu.VMEM((2, SUP, CH), jnp.int32),
            pltpu.VMEM((tail_ch, CH), jnp.int32),
            pltpu.VMEM((N_PAD // 128, 128), jnp.float32),
            pltpu.SemaphoreType.DMA,
        ],
        compiler_params=pltpu.CompilerParams(needs_layout_passes=False),
    )
    def k(dst_hbm, dtail_hbm, out_hbm, didx, dtail, hist, ksem):
        c = lax.axis_index("c")
        s = lax.axis_index("s")
        t = c * NS + s
        n_mine = base_sup + jnp.where(t < extra_sup, 1, 0)

        zv = jnp.zeros((LANES,), jnp.float32)

        def zfill(r, _):
            for j in range(128 // LANES):
                hist[r, pl.ds(j * LANES, LANES)] = zv
            return 0

        lax.fori_loop(0, N_PAD // 128, zfill, 0)

        ov = jnp.ones((LANES,), jnp.float32)

        def scatter_rows(idx_ref2d, j):
            for v in range(CH // LANES):
                idx = idx_ref2d[j, pl.ds(v * LANES, LANES)]
                ihi = lax.shift_right_logical(idx, 7)
                ilo = lax.bitwise_and(idx, 127)
                plsc.addupdate_scatter(hist, [ihi, ilo], ov)

        @pl.when(n_mine > 0)
        def _():
            pltpu.async_copy(dst_hbm.at[pl.ds(t * SUP, SUP)], didx.at[0], ksem)

        def sup_body(u, _):
            b = lax.rem(u, 2)
            pltpu.make_async_copy(dst_hbm.at[pl.ds(0, SUP)], didx.at[b],
                                  ksem).wait()

            @pl.when(u + 1 < n_mine)
            def _():
                row = (t + (u + 1) * NT) * SUP
                pltpu.async_copy(dst_hbm.at[pl.ds(row, SUP)], didx.at[1 - b],
                                 ksem)

            for j in range(SUP):
                scatter_rows(didx.at[b], j)
            return 0

        lax.fori_loop(0, n_mine, sup_body, 0)

        if tail_ch:
            @pl.when(t < tail_ch)
            def _():
                pltpu.sync_copy(dtail_hbm, dtail)
                scatter_rows(dtail, t)

        pltpu.sync_copy(hist, out_hbm.at[t])

    return k(dst2, dst_tail)


def _tc_dense(x, w, hist):
    n = x.shape[0]
    d_out = w.shape[1]

    def body(x_ref, w_ref, hist_ref, g_ref, recip_ref):
        deg = jnp.sum(hist_ref[...], axis=0).reshape(N_PAD) + 1.0
        dis_col = lax.rsqrt(deg)[:, None]
        recip_ref[...] = (1.0 / deg)[:, None]
        h = jnp.dot(x_ref[...], w_ref[...], preferred_element_type=jnp.float32)
        g_ref[...] = h * dis_col[0:n]

    return pl.pallas_call(
        body,
        out_shape=[
            jax.ShapeDtypeStruct((n, d_out), jnp.float32),
            jax.ShapeDtypeStruct((N_PAD, 1), jnp.float32),
        ],
    )(x, w, hist)


def _sc_aggregate(g, src2, dst2, src_tail, dst_tail, zeros_rows):
    n_ch = src2.shape[0] - (src2.shape[0] % SUP)
    n_sup = n_ch // SUP
    base_sup, extra_sup = divmod(n_sup, NT)
    tail_ch = src_tail.shape[0]

    @functools.partial(
        pl.kernel,
        out_type=jax.ShapeDtypeStruct((NC, N_NODES, D), jnp.float32),
        mesh=_MESH,
        scratch_types=[
            pltpu.VMEM((2, SUP, CH), jnp.int32),
            pltpu.VMEM((2, SUP, CH), jnp.int32),
            pltpu.VMEM((2, CH, D), jnp.float32),
            pltpu.VMEM((tail_ch, CH), jnp.int32),
            pltpu.VMEM((tail_ch, CH), jnp.int32),
            pltpu.VMEM_SHARED((N_NODES, D), jnp.float32),
            pltpu.SemaphoreType.DMA,
            pltpu.SemaphoreType.DMA,
            pltpu.SemaphoreType.DMA,
            pltpu.SemaphoreType.DMA,
            pltpu.SemaphoreType.DMA,
        ],
    )
    def k(g_hbm, src_hbm, dst_hbm, stail_hbm, dtail_hbm, zer_hbm, out_hbm,
          sidx, didx, rows, stail, dtail, acc_sp, ksem,
          gsem0, gsem1, ssem0, ssem1):
        c = lax.axis_index("c")
        s = lax.axis_index("s")
        t = c * NS + s
        n_mine = base_sup + jnp.where(t < extra_sup, 1, 0)
        gsems = (gsem0, gsem1)
        ssems = (ssem0, ssem1)

        r0 = s * ROW_STEP

        @pl.when(c == 0)
        def _():
            pltpu.sync_copy(g_hbm.at[pl.ds(r0, ROW_BUF)],
                            acc_sp.at[pl.ds(r0, ROW_BUF)])

        @pl.when(c == 1)
        def _():
            pltpu.sync_copy(zer_hbm, acc_sp.at[pl.ds(r0, ROW_BUF)])

        plsc.subcore_barrier()

        @pl.when(n_mine > 0)
        def _():
            pltpu.async_copy(src_hbm.at[pl.ds(t * SUP, SUP)], sidx.at[0], ksem)
            pltpu.async_copy(dst_hbm.at[pl.ds(t * SUP, SUP)], didx.at[0], ksem)
            pltpu.make_async_copy(src_hbm.at[pl.ds(0, SUP)], sidx.at[0],
                                  ksem).wait()
            pltpu.make_async_copy(dst_hbm.at[pl.ds(0, SUP)], didx.at[0],
                                  ksem).wait()
            pltpu.async_copy(g_hbm.at[sidx.at[0, 0]], rows.at[0], gsem0)

        def sup_body(u, _):
            b = lax.rem(u, 2)

            @pl.when(u + 1 < n_mine)
            def _():
                row = (t + (u + 1) * NT) * SUP
                pltpu.async_copy(src_hbm.at[pl.ds(row, SUP)], sidx.at[1 - b],
                                 ksem)
                pltpu.async_copy(dst_hbm.at[pl.ds(row, SUP)], didx.at[1 - b],
                                 ksem)

            gdesc = [None, None]
            sdesc = [None, None]
            for j in range(SUP):
                r2 = j & 1
                n2 = 1 - r2
                if j == 0:
                    @pl.when(u > 0)
                    def _():
                        pltpu.make_async_copy(
                            rows.at[1], acc_sp.at[didx.at[b, 0]],
                            ssems[1]).wait()
                else:
                    sdesc[n2].wait()
                if j + 1 < SUP:
                    gdesc[n2] = pltpu.async_copy(
                        g_hbm.at[sidx.at[b, j + 1]], rows.at[n2], gsems[n2])
                if j == 0:
                    pltpu.make_async_copy(g_hbm.at[sidx.at[b, 0]],
                                          rows.at[0], gsems[0]).wait()
                else:
                    gdesc[r2].wait()
                sdesc[r2] = pltpu.async_copy(
                    rows.at[r2], acc_sp.at[didx.at[b, j]], ssems[r2],
                    add=True)

            @pl.when(u + 1 < n_mine)
            def _():
                pltpu.make_async_copy(src_hbm.at[pl.ds(0, SUP)],
                                      sidx.at[1 - b], ksem).wait()
                pltpu.make_async_copy(dst_hbm.at[pl.ds(0, SUP)],
                                      didx.at[1 - b], ksem).wait()
                pltpu.async_copy(g_hbm.at[sidx.at[1 - b, 0]], rows.at[0],
                                 gsem0)
            return 0

        lax.fori_loop(0, n_mine, sup_body, 0)
        pltpu.make_async_copy(rows.at[1], acc_sp.at[didx.at[0, 0]],
                              ssems[1]).wait()

        if tail_ch:
            @pl.when(t < tail_ch)
            def _():
                pltpu.sync_copy(stail_hbm, stail)
                pltpu.sync_copy(dtail_hbm, dtail)
                pltpu.async_copy(g_hbm.at[stail.at[t]], rows.at[0],
                                 gsem0).wait()
                pltpu.sync_copy(rows.at[0], acc_sp.at[dtail.at[t]], add=True)

        plsc.subcore_barrier()

        pltpu.sync_copy(acc_sp.at[pl.ds(r0, ROW_BUF)],
                        out_hbm.at[c].at[pl.ds(r0, ROW_BUF)])

    return k(g, src2, dst2, src_tail, dst_tail, zeros_rows)


def _tc_finalize(acc2, recip, b):
    _, n, d = acc2.shape

    def body(a_ref, r_ref, b_ref, o_ref):
        v = (a_ref[0] + a_ref[1]) * r_ref[0:n] + b_ref[...][None, :]
        o_ref[...] = v / (1.0 + jnp.exp(-v))

    return pl.pallas_call(
        body,
        out_shape=jax.ShapeDtypeStruct((n, d), jnp.float32),
    )(acc2, recip, b)


def kernel(x, edge_index, W, b):
    src2 = edge_index[0].astype(jnp.int32).reshape(-1, CH)
    dst2 = edge_index[1].astype(jnp.int32).reshape(-1, CH)
    n_full = (src2.shape[0] // SUP) * SUP
    src_tail = src2[n_full:]
    dst_tail = dst2[n_full:]
    hist = _sc_degree(dst2, dst_tail)
    g, recip = _tc_dense(x, W, hist)
    acc2 = _sc_aggregate(g, src2, dst2, src_tail, dst_tail,
                         jnp.zeros((ROW_BUF, D), jnp.float32))
    return _tc_finalize(acc2, recip, b)
